# Initial kernel scaffold; baseline (speedup 1.0000x reference)
import jax
import jax.numpy as jnp
from jax.experimental import pallas as pl


def kernel(x, edge_index, batch, W1, b1, W2, b2, W3, b3, W4, b4, Wfc1, bfc1, Wfc2, bfc2):
    raise NotImplementedError("write your pallas kernel here")


# speedup vs baseline: 9.9082x; 9.9082x over previous
import jax
import jax.numpy as jnp
from jax import lax
from jax.experimental import pallas as pl
from jax.experimental.pallas import tpu as pltpu
from jax.experimental.pallas import tpu_sc as plsc

N = 10000
E = 320000
D = 128
H = 8
NCLS = 10
SLOPE = 0.01

NCORE = 2
NSUB = 16
NW = NCORE * NSUB
EW = E // NW
CH = 80
NCH = EW // CH
RPS = 624
TAIL = N - NSUB * RPS


def _leaky(v):
    return jnp.where(v >= 0, v, SLOPE * v)


def _mm_body(x_ref, w_ref, o_ref):
    o_ref[:] = jnp.dot(x_ref[:], w_ref[:], preferred_element_type=jnp.float32)


def _mlp_body(t_ref, p_ref, b1_ref, w2_ref, b2_ref, w3_ref, g_ref):
    z = _leaky(t_ref[:] + p_ref[0] + p_ref[1] + b1_ref[:])
    h1 = _leaky(jnp.dot(z, w2_ref[:], preferred_element_type=jnp.float32) + b2_ref[:])
    g_ref[:] = jnp.dot(h1, w3_ref[:], preferred_element_type=jnp.float32)


def _head_body(g_ref, p_ref, b3_ref, w4_ref, b4_ref, wfc1_ref, bfc1_ref,
               wfc2_ref, bfc2_ref, o_ref):
    z = _leaky(g_ref[:] + p_ref[0] + p_ref[1] + b3_ref[:])
    h2 = jnp.dot(z, w4_ref[:], preferred_element_type=jnp.float32) + b4_ref[:]
    q = jnp.dot(_leaky(h2), wfc1_ref[:], preferred_element_type=jnp.float32) + bfc1_ref[:]
    s = _leaky(q)
    r = jnp.sum(s * wfc2_ref[:], axis=0, keepdims=True) + bfc2_ref[:]
    m = jnp.max(r, axis=1, keepdims=True)
    o_ref[:] = r - m - jnp.log(jnp.sum(jnp.exp(r - m), axis=1, keepdims=True))


def _sc_scatter_body(t_hbm, src_hbm, dst_hbm, zero_hbm, out_hbm,
                     src_v, dst_v, rows_v, acc_sh, sem):
    cid = lax.axis_index("c")
    sid = lax.axis_index("s")
    wid = cid * NSUB + sid
    pltpu.sync_copy(zero_hbm.at[pl.ds(sid * RPS, RPS)],
                    acc_sh.at[pl.ds(sid * RPS, RPS)])

    @pl.when(sid == NSUB - 1)
    def _zero_tail():
        pltpu.sync_copy(zero_hbm.at[pl.ds(NSUB * RPS, TAIL)],
                        acc_sh.at[pl.ds(NSUB * RPS, TAIL)])

    pltpu.sync_copy(src_hbm.at[wid], src_v)
    pltpu.sync_copy(dst_hbm.at[wid], dst_v)
    plsc.subcore_barrier()

    def chunk(j, carry):
        pltpu.async_copy(t_hbm.at[src_v.at[j]], rows_v, sem).wait()
        pltpu.sync_copy(rows_v, acc_sh.at[dst_v.at[j]], add=True)
        return carry

    lax.fori_loop(0, NCH, chunk, 0)
    plsc.subcore_barrier()
    pltpu.sync_copy(acc_sh.at[pl.ds(sid * RPS, RPS)],
                    out_hbm.at[cid, pl.ds(sid * RPS, RPS)])

    @pl.when(sid == NSUB - 1)
    def _write_tail():
        pltpu.sync_copy(acc_sh.at[pl.ds(NSUB * RPS, TAIL)],
                        out_hbm.at[cid, pl.ds(NSUB * RPS, TAIL)])


_sc_scatter = pl.kernel(
    _sc_scatter_body,
    out_type=jax.ShapeDtypeStruct((NCORE, N, H), jnp.float32),
    mesh=plsc.VectorSubcoreMesh(core_axis_name="c", subcore_axis_name="s",
                                num_cores=NCORE, num_subcores=NSUB),
    scratch_types=[
        pltpu.VMEM((NCH, CH), jnp.int32),
        pltpu.VMEM((NCH, CH), jnp.int32),
        pltpu.VMEM((CH, H), jnp.float32),
        pltpu.VMEM_SHARED((N, H), jnp.float32),
        pltpu.SemaphoreType.DMA,
    ],
    compiler_params=pltpu.CompilerParams(use_tc_tiling_on_sc=False),
)


def kernel(x, edge_index, batch, W1, b1, W2, b2, W3, b3, W4, b4,
           Wfc1, bfc1, Wfc2, bfc2):
    src = edge_index[0].reshape(NW, NCH, CH)
    dst = edge_index[1].reshape(NW, NCH, CH)
    zero = jnp.zeros((N, H), jnp.float32)

    t1 = pl.pallas_call(
        _mm_body,
        out_shape=jax.ShapeDtypeStruct((N, H), jnp.float32),
    )(x, W1)

    p1 = _sc_scatter(t1, src, dst, zero)

    g = pl.pallas_call(
        _mlp_body,
        out_shape=jax.ShapeDtypeStruct((N, H), jnp.float32),
    )(t1, p1, b1.reshape(1, H), W2, b2.reshape(1, H), W3)

    p2 = _sc_scatter(g, src, dst, zero)

    out = pl.pallas_call(
        _head_body,
        out_shape=jax.ShapeDtypeStruct((1, NCLS), jnp.float32),
    )(g, p2, b3.reshape(1, H), W4, b4.reshape(1, H),
      Wfc1, bfc1.reshape(1, 1), Wfc2, bfc2.reshape(1, NCLS))

    return out


# speedup vs baseline: 21.3597x; 2.1558x over previous
import jax
import jax.numpy as jnp
from jax import lax
from jax.experimental import pallas as pl
from jax.experimental.pallas import tpu as pltpu
from jax.experimental.pallas import tpu_sc as plsc

N = 10000
E = 320000
D = 128
H = 8
NCLS = 10
SLOPE = 0.01

NCORE = 2
NSUB = 16
NW = NCORE * NSUB
EW = E // NW
CH = 2000
NCH = EW // CH
RPS = 624
TAIL = N - NSUB * RPS


def _leaky(v):
    return jnp.where(v >= 0, v, SLOPE * v)


def _mm_body(x_ref, w_ref, o_ref):
    o_ref[:] = jnp.dot(x_ref[:], w_ref[:], preferred_element_type=jnp.float32)


def _mlp_body(t_ref, p_ref, b1_ref, w2_ref, b2_ref, w3_ref, g_ref):
    z = _leaky(t_ref[:] + p_ref[0] + p_ref[1] + b1_ref[:])
    h1 = _leaky(jnp.dot(z, w2_ref[:], preferred_element_type=jnp.float32) + b2_ref[:])
    g_ref[:] = jnp.dot(h1, w3_ref[:], preferred_element_type=jnp.float32)


def _head_body(g_ref, p_ref, b3_ref, w4_ref, b4_ref, wfc1_ref, bfc1_ref,
               wfc2_ref, bfc2_ref, o_ref):
    z = _leaky(g_ref[:] + p_ref[0] + p_ref[1] + b3_ref[:])
    h2 = jnp.dot(z, w4_ref[:], preferred_element_type=jnp.float32) + b4_ref[:]
    q = jnp.dot(_leaky(h2), wfc1_ref[:], preferred_element_type=jnp.float32) + bfc1_ref[:]
    s = _leaky(q)
    r = jnp.sum(s * wfc2_ref[:], axis=0, keepdims=True) + bfc2_ref[:]
    m = jnp.max(r, axis=1, keepdims=True)
    o_ref[:] = r - m - jnp.log(jnp.sum(jnp.exp(r - m), axis=1, keepdims=True))


def _sc_scatter_body(t_hbm, src_hbm, dst_hbm, zero_hbm, out_hbm,
                     src_v, dst_v, rows_v, acc_sh, sem):
    cid = lax.axis_index("c")
    sid = lax.axis_index("s")
    wid = cid * NSUB + sid
    pltpu.sync_copy(zero_hbm.at[pl.ds(sid * RPS, RPS)],
                    acc_sh.at[pl.ds(sid * RPS, RPS)])

    @pl.when(sid == NSUB - 1)
    def _zero_tail():
        pltpu.sync_copy(zero_hbm.at[pl.ds(NSUB * RPS, TAIL)],
                        acc_sh.at[pl.ds(NSUB * RPS, TAIL)])

    pltpu.sync_copy(src_hbm.at[wid], src_v)
    pltpu.sync_copy(dst_hbm.at[wid], dst_v)
    plsc.subcore_barrier()

    def chunk(j, carry):
        pltpu.async_copy(t_hbm.at[src_v.at[j]], rows_v, sem).wait()
        pltpu.sync_copy(rows_v, acc_sh.at[dst_v.at[j]], add=True)
        return carry

    lax.fori_loop(0, NCH, chunk, 0)
    plsc.subcore_barrier()
    pltpu.sync_copy(acc_sh.at[pl.ds(sid * RPS, RPS)],
                    out_hbm.at[cid, pl.ds(sid * RPS, RPS)])

    @pl.when(sid == NSUB - 1)
    def _write_tail():
        pltpu.sync_copy(acc_sh.at[pl.ds(NSUB * RPS, TAIL)],
                        out_hbm.at[cid, pl.ds(NSUB * RPS, TAIL)])


_sc_scatter = pl.kernel(
    _sc_scatter_body,
    out_type=jax.ShapeDtypeStruct((NCORE, N, H), jnp.float32),
    mesh=plsc.VectorSubcoreMesh(core_axis_name="c", subcore_axis_name="s",
                                num_cores=NCORE, num_subcores=NSUB),
    scratch_types=[
        pltpu.VMEM((NCH, CH), jnp.int32),
        pltpu.VMEM((NCH, CH), jnp.int32),
        pltpu.VMEM((CH, H), jnp.float32),
        pltpu.VMEM_SHARED((N, H), jnp.float32),
        pltpu.SemaphoreType.DMA,
    ],
    compiler_params=pltpu.CompilerParams(use_tc_tiling_on_sc=False),
)


def kernel(x, edge_index, batch, W1, b1, W2, b2, W3, b3, W4, b4,
           Wfc1, bfc1, Wfc2, bfc2):
    src = edge_index[0].reshape(NW, NCH, CH)
    dst = edge_index[1].reshape(NW, NCH, CH)
    zero = jnp.zeros((N, H), jnp.float32)

    t1 = pl.pallas_call(
        _mm_body,
        out_shape=jax.ShapeDtypeStruct((N, H), jnp.float32),
    )(x, W1)

    p1 = _sc_scatter(t1, src, dst, zero)

    g = pl.pallas_call(
        _mlp_body,
        out_shape=jax.ShapeDtypeStruct((N, H), jnp.float32),
    )(t1, p1, b1.reshape(1, H), W2, b2.reshape(1, H), W3)

    p2 = _sc_scatter(g, src, dst, zero)

    out = pl.pallas_call(
        _head_body,
        out_shape=jax.ShapeDtypeStruct((1, NCLS), jnp.float32),
    )(g, p2, b3.reshape(1, H), W4, b4.reshape(1, H),
      Wfc1, bfc1.reshape(1, 1), Wfc2, bfc2.reshape(1, NCLS))

    return out


# speedup vs baseline: 21.9190x; 1.0262x over previous
import jax
import jax.numpy as jnp
from jax import lax
from jax.experimental import pallas as pl
from jax.experimental.pallas import tpu as pltpu
from jax.experimental.pallas import tpu_sc as plsc

N = 10000
E = 320000
D = 128
H = 8
NCLS = 10
SLOPE = 0.01

NCORE = 2
NSUB = 16
NW = NCORE * NSUB
EW = E // NW
CH = 10000
NCH = EW // CH
RPS = 624
TAIL = N - NSUB * RPS


def _leaky(v):
    return jnp.where(v >= 0, v, SLOPE * v)


def _mm_body(x_ref, w_ref, o_ref):
    o_ref[:] = jnp.dot(x_ref[:], w_ref[:], preferred_element_type=jnp.float32)


def _mlp_body(t_ref, p_ref, b1_ref, w2_ref, b2_ref, w3_ref, g_ref):
    z = _leaky(t_ref[:] + p_ref[0] + p_ref[1] + b1_ref[:])
    h1 = _leaky(jnp.dot(z, w2_ref[:], preferred_element_type=jnp.float32) + b2_ref[:])
    g_ref[:] = jnp.dot(h1, w3_ref[:], preferred_element_type=jnp.float32)


def _head_body(g_ref, p_ref, b3_ref, w4_ref, b4_ref, wfc1_ref, bfc1_ref,
               wfc2_ref, bfc2_ref, o_ref):
    z = _leaky(g_ref[:] + p_ref[0] + p_ref[1] + b3_ref[:])
    h2 = jnp.dot(z, w4_ref[:], preferred_element_type=jnp.float32) + b4_ref[:]
    q = jnp.dot(_leaky(h2), wfc1_ref[:], preferred_element_type=jnp.float32) + bfc1_ref[:]
    s = _leaky(q)
    r = jnp.sum(s * wfc2_ref[:], axis=0, keepdims=True) + bfc2_ref[:]
    m = jnp.max(r, axis=1, keepdims=True)
    o_ref[:] = r - m - jnp.log(jnp.sum(jnp.exp(r - m), axis=1, keepdims=True))


def _sc_scatter_body(t_hbm, src_hbm, dst_hbm, zero_hbm, out_hbm,
                     src_v, dst_v, rows_v, acc_sh, sem):
    cid = lax.axis_index("c")
    sid = lax.axis_index("s")
    wid = cid * NSUB + sid
    pltpu.sync_copy(zero_hbm.at[pl.ds(sid * RPS, RPS)],
                    acc_sh.at[pl.ds(sid * RPS, RPS)])

    @pl.when(sid == NSUB - 1)
    def _zero_tail():
        pltpu.sync_copy(zero_hbm.at[pl.ds(NSUB * RPS, TAIL)],
                        acc_sh.at[pl.ds(NSUB * RPS, TAIL)])

    pltpu.sync_copy(src_hbm.at[wid], src_v)
    pltpu.sync_copy(dst_hbm.at[wid], dst_v)
    plsc.subcore_barrier()

    def chunk(j, carry):
        pltpu.async_copy(t_hbm.at[src_v.at[j]], rows_v, sem).wait()
        pltpu.sync_copy(rows_v, acc_sh.at[dst_v.at[j]], add=True)
        return carry

    lax.fori_loop(0, NCH, chunk, 0)
    plsc.subcore_barrier()
    pltpu.sync_copy(acc_sh.at[pl.ds(sid * RPS, RPS)],
                    out_hbm.at[cid, pl.ds(sid * RPS, RPS)])

    @pl.when(sid == NSUB - 1)
    def _write_tail():
        pltpu.sync_copy(acc_sh.at[pl.ds(NSUB * RPS, TAIL)],
                        out_hbm.at[cid, pl.ds(NSUB * RPS, TAIL)])


_sc_scatter = pl.kernel(
    _sc_scatter_body,
    out_type=jax.ShapeDtypeStruct((NCORE, N, H), jnp.float32),
    mesh=plsc.VectorSubcoreMesh(core_axis_name="c", subcore_axis_name="s",
                                num_cores=NCORE, num_subcores=NSUB),
    scratch_types=[
        pltpu.VMEM((NCH, CH), jnp.int32),
        pltpu.VMEM((NCH, CH), jnp.int32),
        pltpu.VMEM((CH, H), jnp.float32),
        pltpu.VMEM_SHARED((N, H), jnp.float32),
        pltpu.SemaphoreType.DMA,
    ],
    compiler_params=pltpu.CompilerParams(use_tc_tiling_on_sc=False),
)


def kernel(x, edge_index, batch, W1, b1, W2, b2, W3, b3, W4, b4,
           Wfc1, bfc1, Wfc2, bfc2):
    src = edge_index[0].reshape(NW, NCH, CH)
    dst = edge_index[1].reshape(NW, NCH, CH)
    zero = jnp.zeros((N, H), jnp.float32)

    t1 = pl.pallas_call(
        _mm_body,
        out_shape=jax.ShapeDtypeStruct((N, H), jnp.float32),
    )(x, W1)

    p1 = _sc_scatter(t1, src, dst, zero)

    g = pl.pallas_call(
        _mlp_body,
        out_shape=jax.ShapeDtypeStruct((N, H), jnp.float32),
    )(t1, p1, b1.reshape(1, H), W2, b2.reshape(1, H), W3)

    p2 = _sc_scatter(g, src, dst, zero)

    out = pl.pallas_call(
        _head_body,
        out_shape=jax.ShapeDtypeStruct((1, NCLS), jnp.float32),
    )(g, p2, b3.reshape(1, H), W4, b4.reshape(1, H),
      Wfc1, bfc1.reshape(1, 1), Wfc2, bfc2.reshape(1, NCLS))

    return out


# speedup vs baseline: 22.7889x; 1.0397x over previous
import jax
import jax.numpy as jnp
from jax import lax
from jax.experimental import pallas as pl
from jax.experimental.pallas import tpu as pltpu
from jax.experimental.pallas import tpu_sc as plsc

N = 10000
E = 320000
D = 128
H = 8
NCLS = 10
SLOPE = 0.01

NCORE = 2
NSUB = 16
NW = NCORE * NSUB
EW = E // NW
CH = 5000
NCH = EW // CH
RPS = 624
TAIL = N - NSUB * RPS


def _leaky(v):
    return jnp.where(v >= 0, v, SLOPE * v)


def _mm_body(x_ref, w_ref, o_ref):
    o_ref[:] = jnp.dot(x_ref[:], w_ref[:], preferred_element_type=jnp.float32)


def _mlp_body(t_ref, p_ref, b1_ref, w2_ref, b2_ref, w3_ref, g_ref):
    z = _leaky(t_ref[:] + p_ref[0] + p_ref[1] + b1_ref[:])
    h1 = _leaky(jnp.dot(z, w2_ref[:], preferred_element_type=jnp.float32) + b2_ref[:])
    g_ref[:] = jnp.dot(h1, w3_ref[:], preferred_element_type=jnp.float32)


def _head_body(g_ref, p_ref, b3_ref, w4_ref, b4_ref, wfc1_ref, bfc1_ref,
               wfc2_ref, bfc2_ref, o_ref):
    z = _leaky(g_ref[:] + p_ref[0] + p_ref[1] + b3_ref[:])
    h2 = jnp.dot(z, w4_ref[:], preferred_element_type=jnp.float32) + b4_ref[:]
    q = jnp.dot(_leaky(h2), wfc1_ref[:], preferred_element_type=jnp.float32) + bfc1_ref[:]
    s = _leaky(q)
    r = jnp.sum(s * wfc2_ref[:], axis=0, keepdims=True) + bfc2_ref[:]
    m = jnp.max(r, axis=1, keepdims=True)
    o_ref[:] = r - m - jnp.log(jnp.sum(jnp.exp(r - m), axis=1, keepdims=True))


def _sc_scatter_body(t_hbm, edges_hbm, zero_hbm, out_hbm,
                     src_v, dst_v, rows0_v, rows1_v, acc_sh, sem0, sem1):
    cid = lax.axis_index("c")
    sid = lax.axis_index("s")
    wid = cid * NSUB + sid
    pltpu.sync_copy(zero_hbm.at[pl.ds(sid * RPS, RPS)],
                    acc_sh.at[pl.ds(sid * RPS, RPS)])

    @pl.when(sid == NSUB - 1)
    def _zero_tail():
        pltpu.sync_copy(zero_hbm.at[pl.ds(NSUB * RPS, TAIL)],
                        acc_sh.at[pl.ds(NSUB * RPS, TAIL)])

    pltpu.sync_copy(edges_hbm.at[0, wid], src_v)
    pltpu.sync_copy(edges_hbm.at[1, wid], dst_v)
    plsc.subcore_barrier()

    g0 = pltpu.async_copy(t_hbm.at[src_v.at[0]], rows0_v, sem0)
    g1 = pltpu.async_copy(t_hbm.at[src_v.at[1]], rows1_v, sem1)
    g0.wait()
    pltpu.sync_copy(rows0_v, acc_sh.at[dst_v.at[0]], add=True)
    g1.wait()
    pltpu.sync_copy(rows1_v, acc_sh.at[dst_v.at[1]], add=True)
    plsc.subcore_barrier()
    pltpu.sync_copy(acc_sh.at[pl.ds(sid * RPS, RPS)],
                    out_hbm.at[cid, pl.ds(sid * RPS, RPS)])

    @pl.when(sid == NSUB - 1)
    def _write_tail():
        pltpu.sync_copy(acc_sh.at[pl.ds(NSUB * RPS, TAIL)],
                        out_hbm.at[cid, pl.ds(NSUB * RPS, TAIL)])


_sc_scatter = pl.kernel(
    _sc_scatter_body,
    out_type=jax.ShapeDtypeStruct((NCORE, N, H), jnp.float32),
    mesh=plsc.VectorSubcoreMesh(core_axis_name="c", subcore_axis_name="s",
                                num_cores=NCORE, num_subcores=NSUB),
    scratch_types=[
        pltpu.VMEM((NCH, CH), jnp.int32),
        pltpu.VMEM((NCH, CH), jnp.int32),
        pltpu.VMEM((CH, H), jnp.float32),
        pltpu.VMEM((CH, H), jnp.float32),
        pltpu.VMEM_SHARED((N, H), jnp.float32),
        pltpu.SemaphoreType.DMA,
        pltpu.SemaphoreType.DMA,
    ],
    compiler_params=pltpu.CompilerParams(use_tc_tiling_on_sc=False),
)


def kernel(x, edge_index, batch, W1, b1, W2, b2, W3, b3, W4, b4,
           Wfc1, bfc1, Wfc2, bfc2):
    edges = edge_index.reshape(2, NW, NCH, CH)
    zero = jnp.zeros((N, H), jnp.float32)

    t1 = pl.pallas_call(
        _mm_body,
        out_shape=jax.ShapeDtypeStruct((N, H), jnp.float32),
    )(x, W1)

    p1 = _sc_scatter(t1, edges, zero)

    g = pl.pallas_call(
        _mlp_body,
        out_shape=jax.ShapeDtypeStruct((N, H), jnp.float32),
    )(t1, p1, b1.reshape(1, H), W2, b2.reshape(1, H), W3)

    p2 = _sc_scatter(g, edges, zero)

    out = pl.pallas_call(
        _head_body,
        out_shape=jax.ShapeDtypeStruct((1, NCLS), jnp.float32),
    )(g, p2, b3.reshape(1, H), W4, b4.reshape(1, H),
      Wfc1, bfc1.reshape(1, 1), Wfc2, bfc2.reshape(1, NCLS))

    return out
